# Initial kernel scaffold; baseline (speedup 1.0000x reference)
import jax
import jax.numpy as jnp
from jax.experimental import pallas as pl


def kernel(token_ids, weight):
    raise NotImplementedError("write your pallas kernel here")


# speedup vs baseline: 1.4603x; 1.4603x over previous
import functools

import jax
import jax.numpy as jnp
from jax import lax
from jax.experimental import pallas as pl
from jax.experimental.pallas import tpu as pltpu
from jax.experimental.pallas import tpu_sc as plsc

D_MODEL = 64


@functools.partial(jax.jit, static_argnames=("n_batch", "n_seq"))
def _embedding_gather(token_ids, weight, n_batch, n_seq):
    info = plsc.get_sparse_core_info()
    num_workers = info.num_cores * info.num_subcores
    b_per_w = n_batch // num_workers
    mesh = plsc.VectorSubcoreMesh(core_axis_name="c", subcore_axis_name="s")

    @functools.partial(
        pl.kernel,
        mesh=mesh,
        out_type=jax.ShapeDtypeStruct((n_batch, n_seq, D_MODEL), jnp.float32),
        scratch_types=[
            pltpu.VMEM((b_per_w, n_seq), jnp.int32),
            pltpu.VMEM((n_seq, D_MODEL), jnp.float32),
            pltpu.SemaphoreType.DMA,
        ],
        compiler_params=pltpu.CompilerParams(use_tc_tiling_on_sc=False),
    )
    def k(idx_hbm, table_hbm, out_hbm, idx_v, rows_v, sem):
        wid = lax.axis_index("s") * info.num_cores + lax.axis_index("c")
        base = wid * b_per_w
        pltpu.sync_copy(idx_hbm.at[pl.ds(base, b_per_w)], idx_v)

        def body(b, carry):
            pltpu.async_copy(table_hbm.at[idx_v.at[b]], rows_v, sem).wait()
            pltpu.sync_copy(rows_v, out_hbm.at[base + b])
            return carry

        lax.fori_loop(0, b_per_w, body, 0)

    return k(token_ids, weight)


def kernel(token_ids, weight):
    n_batch, n_seq = token_ids.shape
    return _embedding_gather(token_ids, weight, n_batch, n_seq)


# speedup vs baseline: 1.8723x; 1.2821x over previous
import functools

import jax
import jax.numpy as jnp
from jax import lax
from jax.experimental import pallas as pl
from jax.experimental.pallas import tpu as pltpu
from jax.experimental.pallas import tpu_sc as plsc

D_MODEL = 64
CHUNK = 128
NBUF = 8


@functools.partial(jax.jit, static_argnames=("n_total",))
def _embedding_gather(flat_ids, weight, n_total):
    info = plsc.get_sparse_core_info()
    num_workers = info.num_cores * info.num_subcores
    n_per_w = n_total // num_workers
    n_chunks = n_per_w // CHUNK
    n_groups = n_chunks // NBUF
    assert n_chunks % NBUF == 0
    mesh = plsc.VectorSubcoreMesh(core_axis_name="c", subcore_axis_name="s")

    scratch = (
        [pltpu.VMEM((n_per_w,), jnp.int32)]
        + [pltpu.VMEM((CHUNK, D_MODEL), jnp.float32) for _ in range(NBUF)]
        + [pltpu.SemaphoreType.DMA for _ in range(NBUF)]
        + [pltpu.SemaphoreType.DMA for _ in range(NBUF)]
    )

    @functools.partial(
        pl.kernel,
        mesh=mesh,
        out_type=jax.ShapeDtypeStruct((n_total, D_MODEL), jnp.float32),
        scratch_types=scratch,
        compiler_params=pltpu.CompilerParams(use_tc_tiling_on_sc=False),
    )
    def k(idx_hbm, table_hbm, out_hbm, idx_v, *bufs_and_sems):
        rows = bufs_and_sems[:NBUF]
        gsem = bufs_and_sems[NBUF : 2 * NBUF]
        wsem = bufs_and_sems[2 * NBUF : 3 * NBUF]
        wid = lax.axis_index("s") * info.num_cores + lax.axis_index("c")
        base = wid * n_per_w
        pltpu.sync_copy(idx_hbm.at[pl.ds(base, n_per_w)], idx_v)

        def start_gather(g, b):
            pltpu.async_copy(
                table_hbm.at[idx_v.at[pl.ds(g * CHUNK, CHUNK)]], rows[b], gsem[b]
            )

        def wait_gather(g, b):
            pltpu.make_async_copy(
                table_hbm.at[idx_v.at[pl.ds(g * CHUNK, CHUNK)]], rows[b], gsem[b]
            ).wait()

        def start_write(g, b):
            pltpu.async_copy(
                rows[b], out_hbm.at[pl.ds(base + g * CHUNK, CHUNK)], wsem[b]
            )

        def wait_write(g, b):
            pltpu.make_async_copy(
                rows[b], out_hbm.at[pl.ds(base + g * CHUNK, CHUNK)], wsem[b]
            ).wait()

        for b in range(NBUF):
            start_gather(b, b)

        def group(t, carry):
            for b in range(NBUF):
                g = t * NBUF + b
                wait_gather(g, b)
                start_write(g, b)
            for b in range(NBUF):
                g = t * NBUF + b
                wait_write(g, b)
                start_gather(g + NBUF, b)
            return carry

        lax.fori_loop(0, n_groups - 1, group, 0)

        for b in range(NBUF):
            g = (n_groups - 1) * NBUF + b
            wait_gather(g, b)
            start_write(g, b)
        for b in range(NBUF):
            g = (n_groups - 1) * NBUF + b
            wait_write(g, b)

    return k(flat_ids, weight)


def kernel(token_ids, weight):
    b, s = token_ids.shape
    n_total = b * s
    flat_ids = token_ids.reshape(n_total)
    out = _embedding_gather(flat_ids, weight, n_total)
    return out.reshape(b, s, weight.shape[1])
